# baseline (speedup 1.0000x reference)
import jax
import jax.numpy as jnp
from jax.experimental import pallas as pl

N_EMBD_K = 4096
N_EXP_K = 8
TOP_K_K = 2
CAP_K = 640
NT_K = 2048
ROW_K = N_EXP_K * CAP_K
TB_K = 256


def _route_meta_body(x_ref, w_ref, mi_ref, mf_ref, cap_ref):
    x = x_ref[...]
    w = w_ref[...]
    logits = jax.lax.dot_general(
        x, w, (((1,), (1,)), ((), ())), preferred_element_type=jnp.float32
    )
    nt = NT_K
    e = N_EXP_K
    lane = jax.lax.broadcasted_iota(jnp.int32, (nt, e), 1)

    m1 = jnp.max(logits, axis=1, keepdims=True)
    e1 = jnp.min(jnp.where(logits == m1, lane, e), axis=1, keepdims=True)
    sel1 = lane == e1
    masked = jnp.where(sel1, -jnp.inf, logits)
    m2 = jnp.max(masked, axis=1, keepdims=True)
    e2 = jnp.min(jnp.where(masked == m2, lane, e), axis=1, keepdims=True)
    sel2 = lane == e2

    q = jnp.exp(m2 - m1)
    p1 = 1.0 / (1.0 + q)
    p2 = q * p1

    oh1 = sel1.astype(jnp.float32)
    oh2 = sel2.astype(jnp.float32)

    r = jax.lax.broadcasted_iota(jnp.int32, (nt, nt), 0)
    c = jax.lax.broadcasted_iota(jnp.int32, (nt, nt), 1)
    tril = (r > c).astype(jnp.float32)
    r1x = jax.lax.dot_general(
        tril, oh1, (((1,), (0,)), ((), ())), preferred_element_type=jnp.float32
    )
    r2x = jax.lax.dot_general(
        tril, oh2, (((1,), (0,)), ((), ())), preferred_element_type=jnp.float32
    )
    tot1 = jnp.sum(oh1, axis=0, keepdims=True)
    tot2 = jnp.sum(oh2, axis=0, keepdims=True)

    rank1 = jnp.sum(jnp.where(sel1, r1x, 0.0), axis=1, keepdims=True)
    rank1 = rank1.astype(jnp.int32)
    rank2 = jnp.sum(jnp.where(sel2, r2x + tot1, 0.0), axis=1, keepdims=True)
    rank2 = rank2.astype(jnp.int32)

    pos1 = jnp.where(rank1 < CAP_K, e1 * CAP_K + rank1, -1)
    pos2 = jnp.where(rank2 < CAP_K, e2 * CAP_K + rank2, -1)

    mi_ref[...] = jnp.where(lane == 0, pos1, jnp.where(lane == 1, pos2, -1))
    mf_ref[...] = jnp.where(lane == 0, p1, jnp.where(lane == 1, p2, 0.0))
    used = jnp.minimum(tot1 + tot2, float(CAP_K)).astype(jnp.int32)
    cap_ref[...] = used


def _dispatch_body(mi_ref, mf_ref, cb_ref, mk_ref):
    mi = mi_ref[...]
    mf = mf_ref[...]
    pos1 = mi[:, 0:1]
    pos2 = mi[:, 1:2]
    p1 = mf[:, 0:1]
    p2 = mf[:, 1:2]
    j = jax.lax.broadcasted_iota(jnp.int32, (TB_K, ROW_K), 1)
    val = jnp.where(j == pos1, p1, 0.0) + jnp.where(j == pos2, p2, 0.0)
    cb_ref[...] = val
    mk_ref[...] = (val != 0.0).astype(jnp.int8)


def kernel(x, W_router):
    nt = x.shape[0] * x.shape[1]
    x2 = x.reshape(nt, x.shape[2]).astype(jnp.float32)

    mi, mf, used = pl.pallas_call(
        _route_meta_body,
        out_shape=(
            jax.ShapeDtypeStruct((NT_K, N_EXP_K), jnp.int32),
            jax.ShapeDtypeStruct((NT_K, N_EXP_K), jnp.float32),
            jax.ShapeDtypeStruct((1, N_EXP_K), jnp.int32),
        ),
    )(x2, W_router)

    grid = NT_K // TB_K
    cb2, mk2 = pl.pallas_call(
        _dispatch_body,
        grid=(grid,),
        in_specs=[
            pl.BlockSpec((TB_K, N_EXP_K), lambda i: (i, 0)),
            pl.BlockSpec((TB_K, N_EXP_K), lambda i: (i, 0)),
        ],
        out_specs=(
            pl.BlockSpec((TB_K, ROW_K), lambda i: (i, 0)),
            pl.BlockSpec((TB_K, ROW_K), lambda i: (i, 0)),
        ),
        out_shape=(
            jax.ShapeDtypeStruct((NT_K, ROW_K), jnp.float32),
            jax.ShapeDtypeStruct((NT_K, ROW_K), jnp.int8),
        ),
    )(mi, mf)

    used_cap = used.reshape(N_EXP_K)
    cb_weight = cb2.reshape(NT_K, N_EXP_K, CAP_K)
    sec_mask = (mk2 != 0).reshape(NT_K, N_EXP_K, CAP_K)
    return used_cap, cb_weight, sec_mask


# speedup vs baseline: 2.0204x; 2.0204x over previous
import jax
import jax.numpy as jnp
from jax.experimental import pallas as pl

N_EMBD_K = 4096
N_EXP_K = 8
TOP_K_K = 2
CAP_K = 640
NT_K = 2048
ROW_K = N_EXP_K * CAP_K
TB_K = 256


def _route_meta_body(x_ref, w_ref, mi_ref, mf_ref, cap_ref):
    x = x_ref[...]
    w = w_ref[...]
    logits = jax.lax.dot_general(
        x, w, (((1,), (1,)), ((), ())), preferred_element_type=jnp.float32
    )
    nt = NT_K
    e = N_EXP_K
    lane = jax.lax.broadcasted_iota(jnp.int32, (nt, e), 1)

    m1 = jnp.max(logits, axis=1, keepdims=True)
    e1 = jnp.min(jnp.where(logits == m1, lane, e), axis=1, keepdims=True)
    sel1 = lane == e1
    masked = jnp.where(sel1, -jnp.inf, logits)
    m2 = jnp.max(masked, axis=1, keepdims=True)
    e2 = jnp.min(jnp.where(masked == m2, lane, e), axis=1, keepdims=True)
    sel2 = lane == e2

    q = jnp.exp(m2 - m1)
    p1 = 1.0 / (1.0 + q)
    p2 = q * p1

    oh1 = sel1.astype(jnp.float32)
    oh2 = sel2.astype(jnp.float32)

    r = jax.lax.broadcasted_iota(jnp.int32, (nt, nt), 0)
    c = jax.lax.broadcasted_iota(jnp.int32, (nt, nt), 1)
    tril = (r > c).astype(jnp.float32)
    r1x = jax.lax.dot_general(
        tril, oh1, (((1,), (0,)), ((), ())), preferred_element_type=jnp.float32
    )
    r2x = jax.lax.dot_general(
        tril, oh2, (((1,), (0,)), ((), ())), preferred_element_type=jnp.float32
    )
    tot1 = jnp.sum(oh1, axis=0, keepdims=True)
    tot2 = jnp.sum(oh2, axis=0, keepdims=True)

    rank1 = jnp.sum(jnp.where(sel1, r1x, 0.0), axis=1, keepdims=True)
    rank1 = rank1.astype(jnp.int32)
    rank2 = jnp.sum(jnp.where(sel2, r2x + tot1, 0.0), axis=1, keepdims=True)
    rank2 = rank2.astype(jnp.int32)

    pos1 = jnp.where(rank1 < CAP_K, e1 * CAP_K + rank1, -1)
    pos2 = jnp.where(rank2 < CAP_K, e2 * CAP_K + rank2, -1)

    mi_ref[...] = jnp.where(lane == 0, pos1, jnp.where(lane == 1, pos2, -1))
    mf_ref[...] = jnp.where(lane == 0, p1, jnp.where(lane == 1, p2, 0.0))
    used = jnp.minimum(tot1 + tot2, float(CAP_K)).astype(jnp.int32)
    cap_ref[...] = used


def _dispatch_body(mi_ref, mf_ref, cb_ref, mk_ref):
    mi = mi_ref[...]
    mf = mf_ref[...]
    pos1 = mi[:, 0:1, None]
    pos2 = mi[:, 1:2, None]
    p1 = mf[:, 0:1, None]
    p2 = mf[:, 1:2, None]
    shp = (TB_K, N_EXP_K, CAP_K)
    e = jax.lax.broadcasted_iota(jnp.int32, shp, 1)
    c = jax.lax.broadcasted_iota(jnp.int32, shp, 2)
    j = e * CAP_K + c
    val = jnp.where(j == pos1, p1, 0.0) + jnp.where(j == pos2, p2, 0.0)
    cb_ref[...] = val
    mk_ref[...] = (val != 0.0).astype(jnp.int8)


def kernel(x, W_router):
    nt = x.shape[0] * x.shape[1]
    x2 = x.reshape(nt, x.shape[2]).astype(jnp.float32)

    mi, mf, used = pl.pallas_call(
        _route_meta_body,
        out_shape=(
            jax.ShapeDtypeStruct((NT_K, N_EXP_K), jnp.int32),
            jax.ShapeDtypeStruct((NT_K, N_EXP_K), jnp.float32),
            jax.ShapeDtypeStruct((1, N_EXP_K), jnp.int32),
        ),
    )(x2, W_router)

    grid = NT_K // TB_K
    cb_weight, sec_mask = pl.pallas_call(
        _dispatch_body,
        grid=(grid,),
        in_specs=[
            pl.BlockSpec((TB_K, N_EXP_K), lambda i: (i, 0)),
            pl.BlockSpec((TB_K, N_EXP_K), lambda i: (i, 0)),
        ],
        out_specs=(
            pl.BlockSpec((TB_K, N_EXP_K, CAP_K), lambda i: (i, 0, 0)),
            pl.BlockSpec((TB_K, N_EXP_K, CAP_K), lambda i: (i, 0, 0)),
        ),
        out_shape=(
            jax.ShapeDtypeStruct((NT_K, N_EXP_K, CAP_K), jnp.float32),
            jax.ShapeDtypeStruct((NT_K, N_EXP_K, CAP_K), jnp.int8),
        ),
    )(mi, mf)

    used_cap = used.reshape(N_EXP_K)
    sec_mask = sec_mask.view(jnp.bool_)
    return used_cap, cb_weight, sec_mask


# speedup vs baseline: 2.4111x; 1.1934x over previous
import jax
import jax.numpy as jnp
from jax.experimental import pallas as pl
from jax.experimental.pallas import tpu as pltpu

N_EMBD_K = 4096
N_EXP_K = 8
TOP_K_K = 2
CAP_K = 640
NT_K = 2048
ROW_K = N_EXP_K * CAP_K
TB_K = 256
TM_K = 256
NSTEP_K = NT_K // TM_K


def _route_meta_body(
    x_ref, w_ref, mi_ref, mf_ref, cap_ref, c1_ref, c2_ref, sf_ref, s2_ref, si_ref
):
    i = pl.program_id(0)

    @pl.when(i == 0)
    def _init():
        c1_ref[...] = jnp.zeros((1, N_EXP_K), jnp.float32)
        c2_ref[...] = jnp.zeros((1, N_EXP_K), jnp.float32)

    x = x_ref[...]
    w = w_ref[...]
    logits = jax.lax.dot_general(
        x, w, (((1,), (1,)), ((), ())), preferred_element_type=jnp.float32
    )
    tm = TM_K
    e = N_EXP_K
    lane = jax.lax.broadcasted_iota(jnp.int32, (tm, e), 1)

    m1 = jnp.max(logits, axis=1, keepdims=True)
    e1 = jnp.min(jnp.where(logits == m1, lane, e), axis=1, keepdims=True)
    sel1 = lane == e1
    masked = jnp.where(sel1, -jnp.inf, logits)
    m2 = jnp.max(masked, axis=1, keepdims=True)
    e2 = jnp.min(jnp.where(masked == m2, lane, e), axis=1, keepdims=True)
    sel2 = lane == e2

    q = jnp.exp(m2 - m1)
    p1 = 1.0 / (1.0 + q)
    p2 = q * p1

    oh1 = sel1.astype(jnp.float32)
    oh2 = sel2.astype(jnp.float32)

    r = jax.lax.broadcasted_iota(jnp.int32, (tm, tm), 0)
    c = jax.lax.broadcasted_iota(jnp.int32, (tm, tm), 1)
    tril = (r > c).astype(jnp.float32)
    r1x = jax.lax.dot_general(
        tril, oh1, (((1,), (0,)), ((), ())), preferred_element_type=jnp.float32
    )
    r2x = jax.lax.dot_general(
        tril, oh2, (((1,), (0,)), ((), ())), preferred_element_type=jnp.float32
    )
    tot1 = jnp.sum(oh1, axis=0, keepdims=True)
    tot2 = jnp.sum(oh2, axis=0, keepdims=True)

    carry1 = c1_ref[...]
    carry2 = c2_ref[...]
    rank1 = jnp.sum(jnp.where(sel1, r1x + carry1, 0.0), axis=1, keepdims=True)
    rank1 = rank1.astype(jnp.int32)
    pos1 = jnp.where(rank1 < CAP_K, e1 * CAP_K + rank1, -1)
    pref2 = jnp.sum(jnp.where(sel2, r2x + carry2, 0.0), axis=1, keepdims=True)

    sl = pl.ds(i * tm, tm)
    si_ref[sl, :] = jnp.where(lane == 0, pos1, -1)
    sf_ref[sl, :] = jnp.where(
        lane == 0, p1, jnp.where(lane == 1, p2, jnp.where(lane == 2, pref2, 0.0))
    )
    s2_ref[sl, :] = oh2

    c1_ref[...] = carry1 + tot1
    c2_ref[...] = carry2 + tot2

    @pl.when(i == NSTEP_K - 1)
    def _finalize():
        tot1_all = c1_ref[...]
        tot_all = tot1_all + c2_ref[...]
        lane_a = jax.lax.broadcasted_iota(jnp.int32, (NT_K, N_EXP_K), 1)
        sel2_a = s2_ref[...]
        sf_a = sf_ref[...]
        pref2_a = jnp.sum(jnp.where(lane_a == 2, sf_a, 0.0), axis=1, keepdims=True)
        off2 = jnp.sum(sel2_a * tot1_all, axis=1, keepdims=True)
        rank2 = (pref2_a + off2).astype(jnp.int32)
        e2_a = jnp.sum(sel2_a * lane_a.astype(jnp.float32), axis=1, keepdims=True)
        e2_a = e2_a.astype(jnp.int32)
        pos2 = jnp.where(rank2 < CAP_K, e2_a * CAP_K + rank2, -1)
        mi_ref[...] = jnp.where(lane_a == 1, pos2, si_ref[...])
        mf_ref[...] = jnp.where(lane_a == 2, 0.0, sf_a)
        cap_ref[...] = jnp.minimum(tot_all, float(CAP_K)).astype(jnp.int32)


def _dispatch_body(mi_ref, mf_ref, cb_ref, mk_ref):
    mi = mi_ref[...]
    mf = mf_ref[...]
    pos1 = mi[:, 0:1, None]
    pos2 = mi[:, 1:2, None]
    p1 = mf[:, 0:1, None]
    p2 = mf[:, 1:2, None]
    shp = (TB_K, N_EXP_K, CAP_K)
    e = jax.lax.broadcasted_iota(jnp.int32, shp, 1)
    c = jax.lax.broadcasted_iota(jnp.int32, shp, 2)
    j = e * CAP_K + c
    val = jnp.where(j == pos1, p1, 0.0) + jnp.where(j == pos2, p2, 0.0)
    cb_ref[...] = val
    mk_ref[...] = (val != 0.0).astype(jnp.int8)


def kernel(x, W_router):
    nt = x.shape[0] * x.shape[1]
    x2 = x.reshape(nt, x.shape[2]).astype(jnp.float32)

    mi, mf, used = pl.pallas_call(
        _route_meta_body,
        grid=(NSTEP_K,),
        in_specs=[
            pl.BlockSpec((TM_K, N_EMBD_K), lambda i: (i, 0)),
            pl.BlockSpec((N_EXP_K, N_EMBD_K), lambda i: (0, 0)),
        ],
        out_specs=(
            pl.BlockSpec((NT_K, N_EXP_K), lambda i: (0, 0)),
            pl.BlockSpec((NT_K, N_EXP_K), lambda i: (0, 0)),
            pl.BlockSpec((1, N_EXP_K), lambda i: (0, 0)),
        ),
        out_shape=(
            jax.ShapeDtypeStruct((NT_K, N_EXP_K), jnp.int32),
            jax.ShapeDtypeStruct((NT_K, N_EXP_K), jnp.float32),
            jax.ShapeDtypeStruct((1, N_EXP_K), jnp.int32),
        ),
        scratch_shapes=[
            pltpu.VMEM((1, N_EXP_K), jnp.float32),
            pltpu.VMEM((1, N_EXP_K), jnp.float32),
            pltpu.VMEM((NT_K, N_EXP_K), jnp.float32),
            pltpu.VMEM((NT_K, N_EXP_K), jnp.float32),
            pltpu.VMEM((NT_K, N_EXP_K), jnp.int32),
        ],
    )(x2, W_router)

    grid = NT_K // TB_K
    cb_weight, sec_mask = pl.pallas_call(
        _dispatch_body,
        grid=(grid,),
        in_specs=[
            pl.BlockSpec((TB_K, N_EXP_K), lambda i: (i, 0)),
            pl.BlockSpec((TB_K, N_EXP_K), lambda i: (i, 0)),
        ],
        out_specs=(
            pl.BlockSpec((TB_K, N_EXP_K, CAP_K), lambda i: (i, 0, 0)),
            pl.BlockSpec((TB_K, N_EXP_K, CAP_K), lambda i: (i, 0, 0)),
        ),
        out_shape=(
            jax.ShapeDtypeStruct((NT_K, N_EXP_K, CAP_K), jnp.float32),
            jax.ShapeDtypeStruct((NT_K, N_EXP_K, CAP_K), jnp.int8),
        ),
    )(mi, mf)

    used_cap = used.reshape(N_EXP_K)
    sec_mask = sec_mask.view(jnp.bool_)
    return used_cap, cb_weight, sec_mask
